# baseline (speedup 1.0000x reference)
import jax
import jax.numpy as jnp
from jax.experimental import pallas as pl
from jax.experimental.pallas import tpu as pltpu

EXPR_NUM = 80
KEY_JAW = 16
L = 64
C = 32
NB = EXPR_NUM + KEY_JAW
N_HIDDEN = 128

BLOCK = 8192


def _fused_kernel(xyzt_ref, pk_ref, flx_ref, fly_ref, flz_ref, v0_ref, v1_ref,
                  out_ref,
                  mcat_ref, w1_ref, p2_ref, bcol_ref, w1b_ref):
    @pl.when(pl.program_id(0) == 0)
    def _prologue():
        pk = pk_ref[...]
        g0 = jnp.transpose(pk[0:1, :])
        b0 = jnp.transpose(pk[1:2, :])
        g1 = jnp.transpose(pk[2:3, :])
        b1 = jnp.transpose(pk[3:4, :])
        e = jnp.transpose(pk[6:7, :EXPR_NUM])[:, :, None]
        jw = jnp.transpose(pk[7:8, :KEY_JAW])[:, :, None]
        v0 = v0_ref[...]
        inv0 = g0 * jax.lax.rsqrt(jnp.sum(v0 * v0, axis=1, keepdims=True))
        w0 = v0 * inv0
        for a, fl_ref in ((0, flx_ref), (1, fly_ref), (2, flz_ref)):
            fl = fl_ref[...].reshape(NB, L, C)
            a_bs = jnp.sum(fl[:EXPR_NUM] * e, axis=0)
            a_jw = jnp.sum(fl[EXPR_NUM:] * jw, axis=0)
            w0_bs = w0[:, 32 * a:32 * (a + 1)]
            w0_jw = w0[:, 96 + 32 * a:96 + 32 * (a + 1)]
            mcat_ref[:, 64 * a:64 * (a + 1)] = (
                jax.lax.dot_general(w0_bs, a_bs, (((1,), (1,)), ((), ())),
                                    preferred_element_type=jnp.float32)
                + jax.lax.dot_general(w0_jw, a_jw, (((1,), (1,)), ((), ())),
                                      preferred_element_type=jnp.float32))
        v1 = v1_ref[...]
        inv1 = g1 * jax.lax.rsqrt(jnp.sum(v1 * v1, axis=1, keepdims=True))
        w1_ref[...] = v1 * inv1
        v2 = pk[4:5, :]
        w2 = v2 * (pk[5:6, 0:1] * jax.lax.rsqrt(jnp.sum(v2 * v2)))
        p2_ref[0:1, :] = w2
        bcol_ref[:, 0:1] = b0
        bcol_ref[:, 1:2] = b1
        w1b_ref[...] = w1_ref[...].astype(jnp.bfloat16)

    p = jnp.clip(xyzt_ref[...], 0.0, 1.0) * (L - 1)
    offs = jax.lax.broadcasted_iota(jnp.int32, (L, 1), 0).astype(jnp.float32)
    tent = jnp.concatenate(
        [jnp.maximum(1.0 - jnp.abs(p[a:a + 1, :] - offs), 0.0)
         for a in range(3)], axis=0)

    b0c = bcol_ref[:, 0:1]
    b1c = bcol_ref[:, 1:2]
    h = jnp.dot(mcat_ref[...], tent, preferred_element_type=jnp.float32)
    h = jnp.maximum(h + b0c, 0.0).astype(jnp.bfloat16)
    h = jnp.dot(w1b_ref[...], h, preferred_element_type=jnp.float32)
    h = jnp.maximum(h + b1c, 0.0).astype(jnp.bfloat16)

    out_ref[...] = (jnp.dot(p2_ref[0:1, :].astype(jnp.bfloat16), h,
                            preferred_element_type=jnp.float32)
                    + pk_ref[5:6, 1:2])


@jax.jit
def kernel(expr, jaw_quat_weight, xyz, feat_lines_x, feat_lines_y,
           feat_lines_z, v0, g0, b0, v1, g1, b1, v2, g2, b2):
    n = xyz.shape[0]
    xyzt = xyz.T
    pack = jnp.concatenate([
        g0, b0, g1, b1, v2.reshape(-1), g2, b2,
        jnp.zeros((126,), jnp.float32), expr.reshape(-1)[:EXPR_NUM],
        jnp.zeros((48,), jnp.float32), jaw_quat_weight.reshape(-1),
        jnp.zeros((112,), jnp.float32)]).reshape(8, 128)

    grid = (n // BLOCK,)
    const = lambda shape: pl.BlockSpec(shape, lambda i: (0,) * len(shape))

    out = pl.pallas_call(
        _fused_kernel,
        grid=grid,
        in_specs=[
            pl.BlockSpec((3, BLOCK), lambda i: (0, i)),
            const((8, 128)),
            const((NB * L, C)), const((NB * L, C)), const((NB * L, C)),
            const((N_HIDDEN, 6 * C)), const((N_HIDDEN, N_HIDDEN)),
        ],
        out_specs=pl.BlockSpec((1, BLOCK), lambda i: (0, i)),
        out_shape=jax.ShapeDtypeStruct((1, n), jnp.float32),
        scratch_shapes=[
            pltpu.VMEM((N_HIDDEN, 3 * L), jnp.float32),
            pltpu.VMEM((N_HIDDEN, N_HIDDEN), jnp.float32),
            pltpu.VMEM((1, 128), jnp.float32),
            pltpu.VMEM((N_HIDDEN, 2), jnp.float32),
            pltpu.VMEM((N_HIDDEN, N_HIDDEN), jnp.bfloat16),
        ],
    )(xyzt, pack, feat_lines_x.reshape(NB * L, C),
      feat_lines_y.reshape(NB * L, C), feat_lines_z.reshape(NB * L, C),
      v0, v1)
    return out.reshape(n, 1)


# speedup vs baseline: 1.0096x; 1.0096x over previous
import jax
import jax.numpy as jnp
from jax.experimental import pallas as pl
from jax.experimental.pallas import tpu as pltpu

EXPR_NUM = 80
KEY_JAW = 16
L = 64
C = 32
NB = EXPR_NUM + KEY_JAW
N_HIDDEN = 128

BLOCK = 8192


def _fused_kernel(xyzt_ref, pk_ref, flx_ref, fly_ref, flz_ref, v0_ref, v1_ref,
                  out_ref,
                  mcat_ref, w1_ref, p2_ref, bcol_ref, w1b_ref, tb_ref):
    @pl.when(pl.program_id(0) == 0)
    def _prologue():
        pk = pk_ref[...]
        g0 = jnp.transpose(pk[0:1, :])
        b0 = jnp.transpose(pk[1:2, :])
        g1 = jnp.transpose(pk[2:3, :])
        b1 = jnp.transpose(pk[3:4, :])
        e = jnp.transpose(pk[6:7, :EXPR_NUM])[:, :, None]
        jw = jnp.transpose(pk[7:8, :KEY_JAW])[:, :, None]
        v0 = v0_ref[...]
        inv0 = g0 * jax.lax.rsqrt(jnp.sum(v0 * v0, axis=1, keepdims=True))
        w0 = v0 * inv0
        for a, fl_ref in ((0, flx_ref), (1, fly_ref), (2, flz_ref)):
            fl = fl_ref[...].reshape(NB, L, C)
            a_bs = jnp.sum(fl[:EXPR_NUM] * e, axis=0)
            a_jw = jnp.sum(fl[EXPR_NUM:] * jw, axis=0)
            w0_bs = w0[:, 32 * a:32 * (a + 1)]
            w0_jw = w0[:, 96 + 32 * a:96 + 32 * (a + 1)]
            mcat_ref[:, 64 * a:64 * (a + 1)] = (
                jax.lax.dot_general(w0_bs, a_bs, (((1,), (1,)), ((), ())),
                                    preferred_element_type=jnp.float32)
                + jax.lax.dot_general(w0_jw, a_jw, (((1,), (1,)), ((), ())),
                                      preferred_element_type=jnp.float32))
        v1 = v1_ref[...]
        inv1 = g1 * jax.lax.rsqrt(jnp.sum(v1 * v1, axis=1, keepdims=True))
        w1_ref[...] = v1 * inv1
        v2 = pk[4:5, :]
        w2 = v2 * (pk[5:6, 0:1] * jax.lax.rsqrt(jnp.sum(v2 * v2)))
        p2_ref[0:1, :] = w2
        bcol_ref[:, 0:1] = b0
        bcol_ref[:, 1:2] = b1
        w1b_ref[...] = w1_ref[...].astype(jnp.bfloat16)

    i = pl.program_id(0)
    nsteps = pl.num_programs(0) - 1

    @pl.when(i < nsteps)
    def _build_tent():
        p = jnp.clip(xyzt_ref[...], 0.0, 1.0) * (L - 1)
        offs = (jax.lax.broadcasted_iota(jnp.int32, (L, 1), 0)
                .astype(jnp.float32))
        tent = jnp.concatenate(
            [jnp.maximum(1.0 - jnp.abs(p[a:a + 1, :] - offs), 0.0)
             for a in range(3)], axis=0)
        tb_ref[jax.lax.rem(i, 2)] = tent

    @pl.when(i > 0)
    def _mlp():
        tent = tb_ref[jax.lax.rem(i + 1, 2)]
        b0c = bcol_ref[:, 0:1]
        b1c = bcol_ref[:, 1:2]
        h = jnp.dot(mcat_ref[...], tent, preferred_element_type=jnp.float32)
        h = jnp.maximum(h + b0c, 0.0).astype(jnp.bfloat16)
        h = jnp.dot(w1b_ref[...], h, preferred_element_type=jnp.float32)
        h = jnp.maximum(h + b1c, 0.0).astype(jnp.bfloat16)
        out_ref[...] = (jnp.dot(p2_ref[0:1, :].astype(jnp.bfloat16), h,
                                preferred_element_type=jnp.float32)
                        + pk_ref[5:6, 1:2])


@jax.jit
def kernel(expr, jaw_quat_weight, xyz, feat_lines_x, feat_lines_y,
           feat_lines_z, v0, g0, b0, v1, g1, b1, v2, g2, b2):
    n = xyz.shape[0]
    xyzt = xyz.T
    pack = jnp.concatenate([
        g0, b0, g1, b1, v2.reshape(-1), g2, b2,
        jnp.zeros((126,), jnp.float32), expr.reshape(-1)[:EXPR_NUM],
        jnp.zeros((48,), jnp.float32), jaw_quat_weight.reshape(-1),
        jnp.zeros((112,), jnp.float32)]).reshape(8, 128)

    nsteps = n // BLOCK
    grid = (nsteps + 1,)
    const = lambda shape: pl.BlockSpec(shape, lambda i: (0,) * len(shape))

    out = pl.pallas_call(
        _fused_kernel,
        grid=grid,
        in_specs=[
            pl.BlockSpec((3, BLOCK),
                         lambda i: (0, jnp.minimum(i, nsteps - 1))),
            const((8, 128)),
            const((NB * L, C)), const((NB * L, C)), const((NB * L, C)),
            const((N_HIDDEN, 6 * C)), const((N_HIDDEN, N_HIDDEN)),
        ],
        out_specs=pl.BlockSpec((1, BLOCK), lambda i: (0, jnp.maximum(i - 1, 0))),
        out_shape=jax.ShapeDtypeStruct((1, n), jnp.float32),
        scratch_shapes=[
            pltpu.VMEM((N_HIDDEN, 3 * L), jnp.float32),
            pltpu.VMEM((N_HIDDEN, N_HIDDEN), jnp.float32),
            pltpu.VMEM((1, 128), jnp.float32),
            pltpu.VMEM((N_HIDDEN, 2), jnp.float32),
            pltpu.VMEM((N_HIDDEN, N_HIDDEN), jnp.bfloat16),
            pltpu.VMEM((2, 3 * L, BLOCK), jnp.float32),
        ],
    )(xyzt, pack, feat_lines_x.reshape(NB * L, C),
      feat_lines_y.reshape(NB * L, C), feat_lines_z.reshape(NB * L, C),
      v0, v1)
    return out.reshape(n, 1)


# speedup vs baseline: 1.0181x; 1.0084x over previous
import jax
import jax.numpy as jnp
from jax.experimental import pallas as pl
from jax.experimental.pallas import tpu as pltpu

EXPR_NUM = 80
KEY_JAW = 16
L = 64
C = 32
NB = EXPR_NUM + KEY_JAW
N_HIDDEN = 128

BLOCK = 8192


def _fused_kernel(xyzt_ref, pk_ref, flx_ref, fly_ref, flz_ref, v0_ref, v1_ref,
                  out_ref,
                  mcat_ref, w1_ref, p2_ref, bcol_ref, w1b_ref):
    @pl.when(pl.program_id(0) == 0)
    def _prologue():
        pk = pk_ref[...]
        g0 = jnp.transpose(pk[0:1, :])
        b0 = jnp.transpose(pk[1:2, :])
        g1 = jnp.transpose(pk[2:3, :])
        b1 = jnp.transpose(pk[3:4, :])
        e = jnp.transpose(pk[6:7, :EXPR_NUM])[:, :, None]
        jw = jnp.transpose(pk[7:8, :KEY_JAW])[:, :, None]
        v0 = v0_ref[...]
        inv0 = g0 * jax.lax.rsqrt(jnp.sum(v0 * v0, axis=1, keepdims=True))
        w0 = v0 * inv0
        for a, fl_ref in ((0, flx_ref), (1, fly_ref), (2, flz_ref)):
            fl = fl_ref[...].reshape(NB, L, C)
            a_bs = jnp.sum(fl[:EXPR_NUM] * e, axis=0)
            a_jw = jnp.sum(fl[EXPR_NUM:] * jw, axis=0)
            w0_bs = w0[:, 32 * a:32 * (a + 1)]
            w0_jw = w0[:, 96 + 32 * a:96 + 32 * (a + 1)]
            mcat_ref[:, 64 * a:64 * (a + 1)] = (
                jax.lax.dot_general(w0_bs, a_bs, (((1,), (1,)), ((), ())),
                                    preferred_element_type=jnp.float32)
                + jax.lax.dot_general(w0_jw, a_jw, (((1,), (1,)), ((), ())),
                                      preferred_element_type=jnp.float32))
        v1 = v1_ref[...]
        inv1 = g1 * jax.lax.rsqrt(jnp.sum(v1 * v1, axis=1, keepdims=True))
        w1_ref[...] = v1 * inv1
        v2 = pk[4:5, :]
        w2 = v2 * (pk[5:6, 0:1] * jax.lax.rsqrt(jnp.sum(v2 * v2)))
        p2_ref[0:1, :] = w2
        bcol_ref[:, 0:1] = b0
        bcol_ref[:, 1:2] = b1
        w1b_ref[...] = w1_ref[...].astype(jnp.bfloat16)

    offs = jax.lax.broadcasted_iota(jnp.int32, (L, 1), 0).astype(jnp.float32)
    b0c = bcol_ref[:, 0:1]
    b1c = bcol_ref[:, 1:2]
    w2b = p2_ref[0:1, :].astype(jnp.bfloat16)
    HB = BLOCK // 2
    for half in range(2):
        sl = pl.ds(half * HB, HB)
        p = jnp.clip(xyzt_ref[:, sl], 0.0, 1.0) * (L - 1)
        tent = jnp.concatenate(
            [jnp.maximum(1.0 - jnp.abs(p[a:a + 1, :] - offs), 0.0)
             for a in range(3)], axis=0)
        h = jnp.dot(mcat_ref[...], tent, preferred_element_type=jnp.float32)
        h = jnp.maximum(h + b0c, 0.0).astype(jnp.bfloat16)
        h = jnp.dot(w1b_ref[...], h, preferred_element_type=jnp.float32)
        h = jnp.maximum(h + b1c, 0.0).astype(jnp.bfloat16)
        out_ref[:, sl] = (jnp.dot(w2b, h, preferred_element_type=jnp.float32)
                          + pk_ref[5:6, 1:2])


@jax.jit
def kernel(expr, jaw_quat_weight, xyz, feat_lines_x, feat_lines_y,
           feat_lines_z, v0, g0, b0, v1, g1, b1, v2, g2, b2):
    n = xyz.shape[0]
    xyzt = xyz.T
    pack = jnp.concatenate([
        g0, b0, g1, b1, v2.reshape(-1), g2, b2,
        jnp.zeros((126,), jnp.float32), expr.reshape(-1)[:EXPR_NUM],
        jnp.zeros((48,), jnp.float32), jaw_quat_weight.reshape(-1),
        jnp.zeros((112,), jnp.float32)]).reshape(8, 128)

    grid = (n // BLOCK,)
    const = lambda shape: pl.BlockSpec(shape, lambda i: (0,) * len(shape))

    out = pl.pallas_call(
        _fused_kernel,
        grid=grid,
        in_specs=[
            pl.BlockSpec((3, BLOCK), lambda i: (0, i)),
            const((8, 128)),
            const((NB * L, C)), const((NB * L, C)), const((NB * L, C)),
            const((N_HIDDEN, 6 * C)), const((N_HIDDEN, N_HIDDEN)),
        ],
        out_specs=pl.BlockSpec((1, BLOCK), lambda i: (0, i)),
        out_shape=jax.ShapeDtypeStruct((1, n), jnp.float32),
        scratch_shapes=[
            pltpu.VMEM((N_HIDDEN, 3 * L), jnp.float32),
            pltpu.VMEM((N_HIDDEN, N_HIDDEN), jnp.float32),
            pltpu.VMEM((1, 128), jnp.float32),
            pltpu.VMEM((N_HIDDEN, 2), jnp.float32),
            pltpu.VMEM((N_HIDDEN, N_HIDDEN), jnp.bfloat16),
        ],
    )(xyzt, pack, feat_lines_x.reshape(NB * L, C),
      feat_lines_y.reshape(NB * L, C), feat_lines_z.reshape(NB * L, C),
      v0, v1)
    return out.reshape(n, 1)
